# Initial kernel scaffold; baseline (speedup 1.0000x reference)
import jax
import jax.numpy as jnp
from jax.experimental import pallas as pl


def kernel(x, edge_index, W1, b1, W2, b2):
    raise NotImplementedError("write your pallas kernel here")


# speedup vs baseline: 29.6354x; 29.6354x over previous
import jax
import jax.numpy as jnp
from jax import lax
from jax.experimental import pallas as pl
from jax.experimental.pallas import tpu as pltpu
from jax.experimental.pallas import tpu_sc as plsc

N = 10000
E = 320000
DF = 128
DH = 16
NCORES = 2
NSUB = 16
NTILES = NCORES * NSUB
EPT = E // NTILES
W = 80
NWIN = EPT // W
NPAD = 10240
RPT = NPAD // NSUB
DPT = NPAD // NSUB

_mesh = plsc.VectorSubcoreMesh(core_axis_name="c", subcore_axis_name="s")


def _deg_body(dst_ref, out_ref, idx, ones, zbuf, acc):
    cid = lax.axis_index("c")
    sid = lax.axis_index("s")
    wid = cid * NSUB + sid
    for i in range(W // 16):
        ones[pl.ds(i * 16, 16)] = jnp.ones((16,), jnp.float32)

    def zb(i, carry):
        zbuf[pl.ds(i * 16, 16)] = jnp.zeros((16,), jnp.float32)
        return carry

    lax.fori_loop(0, DPT // 16, zb, 0)
    pltpu.sync_copy(zbuf, acc.at[pl.ds(sid * DPT, DPT)])
    plsc.subcore_barrier()
    pltpu.sync_copy(dst_ref.at[wid], idx)

    def body(j, carry):
        pltpu.sync_copy(ones, acc.at[idx.at[j]], add=True)
        return carry

    lax.fori_loop(0, NWIN, body, 0)
    plsc.subcore_barrier()
    pltpu.sync_copy(acc.at[pl.ds(sid * DPT, DPT)],
                    out_ref.at[cid, pl.ds(sid * DPT, DPT)])


_deg_kernel = pl.kernel(
    _deg_body,
    out_type=jax.ShapeDtypeStruct((NCORES, NPAD), jnp.float32),
    mesh=_mesh,
    scratch_types=[
        pltpu.VMEM((NWIN, W), jnp.int32),
        pltpu.VMEM((W,), jnp.float32),
        pltpu.VMEM((DPT,), jnp.float32),
        pltpu.VMEM_SHARED((NPAD,), jnp.float32),
    ],
)


def _agg_body(tab_ref, src_ref, dst_ref, out_ref, idxs, idxd, rows, zbuf, acc):
    cid = lax.axis_index("c")
    sid = lax.axis_index("s")
    wid = cid * NSUB + sid

    def zb(i, carry):
        zbuf[i] = jnp.zeros((16,), jnp.float32)
        return carry

    lax.fori_loop(0, RPT, zb, 0)
    pltpu.sync_copy(zbuf, acc.at[pl.ds(sid * RPT, RPT)])
    plsc.subcore_barrier()
    pltpu.sync_copy(src_ref.at[wid], idxs)
    pltpu.sync_copy(dst_ref.at[wid], idxd)

    def body(j, carry):
        pltpu.sync_copy(tab_ref.at[idxs.at[j]], rows)
        pltpu.sync_copy(rows, acc.at[idxd.at[j]], add=True)
        return carry

    lax.fori_loop(0, NWIN, body, 0)
    plsc.subcore_barrier()
    pltpu.sync_copy(acc.at[pl.ds(sid * RPT, RPT)],
                    out_ref.at[cid, pl.ds(sid * RPT, RPT)])


_agg_kernel = pl.kernel(
    _agg_body,
    out_type=jax.ShapeDtypeStruct((NCORES, NPAD, DH), jnp.float32),
    mesh=_mesh,
    compiler_params=pltpu.CompilerParams(use_tc_tiling_on_sc=False),
    scratch_types=[
        pltpu.VMEM((NWIN, W), jnp.int32),
        pltpu.VMEM((NWIN, W), jnp.int32),
        pltpu.VMEM((W, DH), jnp.float32),
        pltpu.VMEM((RPT, DH), jnp.float32),
        pltpu.VMEM_SHARED((NPAD, DH), jnp.float32),
    ],
)


def _tc1_body(x_ref, w1_ref, degt_ref, xw_ref, z1_ref, dinv_ref):
    deg = jnp.sum(degt_ref[...], axis=1, keepdims=True) + 1.0
    dinv = lax.rsqrt(deg)
    xw = jnp.dot(x_ref[...], w1_ref[...], preferred_element_type=jnp.float32)
    xw_ref[...] = xw
    z1_ref[...] = xw * dinv
    dinv_ref[...] = dinv


def _tc1(x, w1, degt):
    return pl.pallas_call(
        _tc1_body,
        out_shape=[
            jax.ShapeDtypeStruct((N, DH), jnp.float32),
            jax.ShapeDtypeStruct((N, DH), jnp.float32),
            jax.ShapeDtypeStruct((N, 1), jnp.float32),
        ],
    )(x, w1, degt)


def _tc2_body(agg_ref, xw_ref, dinv_ref, w2_ref, b1_ref, hw2_ref, z2_ref):
    dinv = dinv_ref[...]
    agg = agg_ref[0, :N, :] + agg_ref[1, :N, :]
    pre = dinv * agg + dinv * dinv * xw_ref[...] + b1_ref[...]
    h = jnp.maximum(pre, 0.0)
    hw2 = jnp.dot(h, w2_ref[...], preferred_element_type=jnp.float32)
    hw2_ref[...] = hw2
    z2_ref[...] = hw2 * dinv


def _tc2(agg1, xw, dinv, w2, b1):
    return pl.pallas_call(
        _tc2_body,
        out_shape=[
            jax.ShapeDtypeStruct((N, DH), jnp.float32),
            jax.ShapeDtypeStruct((N, DH), jnp.float32),
        ],
    )(agg1, xw, dinv, w2, b1)


def _tc3_body(agg_ref, hw2_ref, dinv_ref, b2_ref, out_ref):
    dinv = dinv_ref[...]
    logits = (dinv * (agg_ref[0, :N, :] + agg_ref[1, :N, :])
              + dinv * dinv * hw2_ref[...] + b2_ref[...])
    m = jnp.max(logits, axis=1, keepdims=True)
    s = jnp.sum(jnp.exp(logits - m), axis=1, keepdims=True)
    out_ref[...] = logits - m - jnp.log(s)


def _tc3(agg2, hw2, dinv, b2):
    return pl.pallas_call(
        _tc3_body,
        out_shape=jax.ShapeDtypeStruct((N, DH), jnp.float32),
    )(agg2, hw2, dinv, b2)


def kernel(x, edge_index, W1, b1, W2, b2):
    src = edge_index[0].astype(jnp.int32).reshape(NTILES, NWIN, W)
    dst = edge_index[1].astype(jnp.int32).reshape(NTILES, NWIN, W)
    degp = _deg_kernel(dst)
    degt = jnp.transpose(degp[:, :N])
    xw, z1, dinv = _tc1(x, W1, degt)
    agg1 = _agg_kernel(z1, src, dst)
    hw2, z2 = _tc2(agg1, xw, dinv, W2, b1.reshape(1, DH))
    agg2 = _agg_kernel(z2, src, dst)
    return _tc3(agg2, hw2, dinv, b2.reshape(1, DH))


# speedup vs baseline: 62.0563x; 2.0940x over previous
import jax
import jax.numpy as jnp
from jax import lax
from jax.experimental import pallas as pl
from jax.experimental.pallas import tpu as pltpu
from jax.experimental.pallas import tpu_sc as plsc

N = 10000
E = 320000
DF = 128
DH = 16
NCORES = 2
NSUB = 16
NTILES = NCORES * NSUB
NPAD = 10240
NROW = NPAD // 16

EPS = E // NSUB
DSTEPS = EPS // 16
NCOMB = NROW // 128

EPC = E // NCORES
CH = 16000
NCH = EPC // CH
STEPS = CH // 16

_mesh = plsc.VectorSubcoreMesh(core_axis_name="c", subcore_axis_name="s")

_SQRT_MAGIC = 0x5F3759DF


def _rsqrt_newton(x):
    i = plsc.bitcast(x, jnp.int32)
    i = _SQRT_MAGIC - lax.shift_right_arithmetic(i, 1)
    y = plsc.bitcast(i, jnp.float32)
    for _ in range(3):
        y = y * (1.5 - 0.5 * x * y * y)
    return y


def _zero_f32(ref, nrows):
    def zb(i, carry):
        ref[pl.ds(i * 16, 16)] = jnp.zeros((16,), jnp.float32)
        return carry

    lax.fori_loop(0, nrows, zb, 0)


def _agg_chunks(edge_ref, ebuf, esem, tab, acc, cid):
    pltpu.async_copy(edge_ref.at[cid, 0], ebuf.at[0], esem.at[0])

    def chunk(k, carry):
        b = lax.rem(k, 2)
        nxt = 1 - b

        @pl.when(k + 1 < NCH)
        def _():
            pltpu.async_copy(edge_ref.at[cid, k + 1], ebuf.at[nxt],
                             esem.at[nxt])

        pltpu.make_async_copy(edge_ref.at[cid, k], ebuf.at[b],
                              esem.at[b]).wait()

        @plsc.parallel_loop(0, STEPS, 1, unroll=8)
        def step(i):
            e = ebuf[b, pl.ds(i * 16, 16)]
            s16 = jnp.bitwise_and(e, 0xFFFF)
            d16 = lax.shift_right_logical(e, 16)
            v = plsc.load_gather(tab, [s16])
            plsc.addupdate_scatter(acc, [d16], v)

        return carry

    lax.fori_loop(0, NCH, chunk, 0)


def _sc1_body(tab_ref, dstp_ref, edge_ref, out_ref, dinv_ref,
              dbuf, ebuf, tab, dloc, acc, hacc, iotab, tsem, dsem, esem,
              degacc):
    cid = lax.axis_index("c")
    sid = lax.axis_index("s")
    tdesc = pltpu.async_copy(tab_ref.at[sid], tab, tsem)
    ddesc = pltpu.async_copy(dstp_ref.at[sid], dbuf, dsem)

    for k in range(NCOMB):
        for c in range(8):
            iotab[k, pl.ds(c * 16, 16)] = (lax.iota(jnp.int32, 16)
                                           + (k * 128 + c * 16))

    def zh(i, carry):
        hacc[i] = jnp.zeros((16,), jnp.float32)
        return carry

    lax.fori_loop(0, NROW, zh, 0)
    pltpu.sync_copy(hacc.at[pl.ds(sid * (NROW // NSUB), NROW // NSUB)],
                    degacc.at[pl.ds(sid * (NROW // NSUB), NROW // NSUB)])
    plsc.subcore_barrier()

    ddesc.wait()
    ones16 = jnp.ones((16,), jnp.float32)

    @plsc.parallel_loop(0, DSTEPS, 1, unroll=8)
    def dstep(i):
        d = dbuf[pl.ds(i * 16, 16)]
        r = lax.shift_right_logical(d, 4)
        c = jnp.bitwise_and(d, 15)
        plsc.addupdate_scatter(hacc, [r, c], ones16)

    def comb(k, carry):
        pltpu.sync_copy(hacc.at[pl.ds(k * 128, 128)],
                        degacc.at[iotab.at[k]], add=True)
        return carry

    lax.fori_loop(0, NCOMB, comb, 0)
    plsc.subcore_barrier()

    pltpu.sync_copy(degacc, dloc)

    def nwt(i, carry):
        y = _rsqrt_newton(dloc[i] + 1.0)
        dloc[i] = y
        return carry

    lax.fori_loop(0, NROW, nwt, 0)

    @pl.when(jnp.logical_and(cid == 0, sid == 0))
    def _():
        pltpu.sync_copy(dloc, dinv_ref)

    tdesc.wait()

    def sc(i, carry):
        tab[pl.ds(i * 16, 16)] = tab[pl.ds(i * 16, 16)] * dloc[i]
        return carry

    lax.fori_loop(0, N // 16, sc, 0)
    _zero_f32(acc, NROW)

    _agg_chunks(edge_ref, ebuf, esem, tab, acc, cid)
    pltpu.sync_copy(acc, out_ref.at[cid, sid])


_sc1_kernel = pl.kernel(
    _sc1_body,
    out_type=(jax.ShapeDtypeStruct((NCORES, DH, NPAD), jnp.float32),
              jax.ShapeDtypeStruct((NROW, 16), jnp.float32)),
    mesh=_mesh,
    compiler_params=pltpu.CompilerParams(use_tc_tiling_on_sc=False,
                                         needs_layout_passes=False),
    scratch_types=[
        pltpu.VMEM((EPS,), jnp.int32),
        pltpu.VMEM((2, CH), jnp.int32),
        pltpu.VMEM((N,), jnp.float32),
        pltpu.VMEM((NROW, 16), jnp.float32),
        pltpu.VMEM((NPAD,), jnp.float32),
        pltpu.VMEM((NROW, 16), jnp.float32),
        pltpu.VMEM((NCOMB, 128), jnp.int32),
        pltpu.SemaphoreType.DMA,
        pltpu.SemaphoreType.DMA,
        pltpu.SemaphoreType.DMA((2,)),
        pltpu.VMEM_SHARED((NROW, 16), jnp.float32),
    ],
)


def _sc2_body(tab_ref, dinv_ref, edge_ref, out_ref,
              ebuf, tab, dloc, acc, tsem, dsem, esem):
    cid = lax.axis_index("c")
    sid = lax.axis_index("s")
    tdesc = pltpu.async_copy(tab_ref.at[sid], tab, tsem)
    ddesc = pltpu.async_copy(dinv_ref, dloc, dsem)
    _zero_f32(acc, NROW)
    tdesc.wait()
    ddesc.wait()

    def sc(i, carry):
        tab[pl.ds(i * 16, 16)] = tab[pl.ds(i * 16, 16)] * dloc[i]
        return carry

    lax.fori_loop(0, N // 16, sc, 0)
    _agg_chunks(edge_ref, ebuf, esem, tab, acc, cid)
    pltpu.sync_copy(acc, out_ref.at[cid, sid])


_sc2_kernel = pl.kernel(
    _sc2_body,
    out_type=jax.ShapeDtypeStruct((NCORES, DH, NPAD), jnp.float32),
    mesh=_mesh,
    compiler_params=pltpu.CompilerParams(use_tc_tiling_on_sc=False,
                                         needs_layout_passes=False),
    scratch_types=[
        pltpu.VMEM((2, CH), jnp.int32),
        pltpu.VMEM((N,), jnp.float32),
        pltpu.VMEM((NROW, 16), jnp.float32),
        pltpu.VMEM((NPAD,), jnp.float32),
        pltpu.SemaphoreType.DMA,
        pltpu.SemaphoreType.DMA,
        pltpu.SemaphoreType.DMA((2,)),
    ],
)


def _tcxw_body(x_ref, w1_ref, xwt_ref):
    xwt_ref[...] = lax.dot_general(w1_ref[...], x_ref[...],
                                   (((0,), (1,)), ((), ())),
                                   preferred_element_type=jnp.float32,
                                   precision=lax.Precision.HIGHEST)


def _tcxw(x, w1):
    return pl.pallas_call(
        _tcxw_body,
        out_shape=jax.ShapeDtypeStruct((DH, N), jnp.float32),
    )(x, w1)


def _tcmid_body(agg_ref, xwt_ref, dinv_ref, w2_ref, b1_ref, hw2t_ref):
    dinv = dinv_ref[0:1, :N]
    agg = agg_ref[0, :, :N] + agg_ref[1, :, :N]
    pre = dinv * agg + dinv * dinv * xwt_ref[...] + b1_ref[...]
    h = jnp.maximum(pre, 0.0)
    hw2t_ref[...] = lax.dot_general(w2_ref[...], h, (((0,), (0,)), ((), ())),
                                    preferred_element_type=jnp.float32,
                                    precision=lax.Precision.HIGHEST)


def _tcmid(agg1, xwt, dinv, w2, b1):
    return pl.pallas_call(
        _tcmid_body,
        out_shape=jax.ShapeDtypeStruct((DH, N), jnp.float32),
    )(agg1, xwt, dinv, w2, b1)


def _tc3_body(agg_ref, hw2t_ref, dinv_ref, b2_ref, eye_ref, out_ref):
    dinv = dinv_ref[0:1, :N]
    lg = (dinv * (agg_ref[0, :, :N] + agg_ref[1, :, :N])
          + dinv * dinv * hw2t_ref[...] + b2_ref[...])
    m = jnp.max(lg, axis=0, keepdims=True)
    s = jnp.sum(jnp.exp(lg - m), axis=0, keepdims=True)
    lsm = lg - m - jnp.log(s)
    out_ref[...] = lax.dot_general(lsm, eye_ref[...], (((0,), (0,)), ((), ())),
                                   preferred_element_type=jnp.float32,
                                   precision=lax.Precision.HIGHEST)


def _tc3(agg2, hw2t, dinv, b2, eye):
    return pl.pallas_call(
        _tc3_body,
        out_shape=jax.ShapeDtypeStruct((N, DH), jnp.float32),
    )(agg2, hw2t, dinv, b2, eye)


def kernel(x, edge_index, W1, b1, W2, b2):
    src = edge_index[0].astype(jnp.int32)
    dst = edge_index[1].astype(jnp.int32)
    epack = jnp.bitwise_or(src, lax.shift_left(dst, 16))
    edges = epack.reshape(NCORES, NCH, CH)
    dstp = dst.reshape(NSUB, EPS)
    xwt = _tcxw(x, W1)
    agg1, dinv = _sc1_kernel(xwt, dstp, edges)
    dinvr = dinv.reshape(1, NPAD)
    hw2t = _tcmid(agg1, xwt, dinvr, W2, b1.reshape(DH, 1))
    agg2 = _sc2_kernel(hw2t, dinv, edges)
    return _tc3(agg2, hw2t, dinvr, b2.reshape(DH, 1),
                jnp.eye(DH, dtype=jnp.float32))


# speedup vs baseline: 65.0537x; 1.0483x over previous
import jax
import jax.numpy as jnp
from jax import lax
from jax.experimental import pallas as pl
from jax.experimental.pallas import tpu as pltpu
from jax.experimental.pallas import tpu_sc as plsc

N = 10000
E = 320000
DF = 128
DH = 16
NCORES = 2
NSUB = 16
NTILES = NCORES * NSUB
NPAD = 10240

EPT = E // NTILES
W = 80
NWIN = EPT // W
DPT = NPAD // NSUB

EPC = E // NCORES
CH = 16000
NCH = EPC // CH
STEPS = CH // 16

_mesh = plsc.VectorSubcoreMesh(core_axis_name="c", subcore_axis_name="s")


def _deg_body(dst_ref, out_ref, idx, ones, zbuf, acc):
    cid = lax.axis_index("c")
    sid = lax.axis_index("s")
    wid = cid * NSUB + sid
    for i in range(W // 16):
        ones[pl.ds(i * 16, 16)] = jnp.ones((16,), jnp.float32)

    def zb(i, carry):
        zbuf[pl.ds(i * 16, 16)] = jnp.zeros((16,), jnp.float32)
        return carry

    lax.fori_loop(0, DPT // 16, zb, 0)
    pltpu.sync_copy(zbuf, acc.at[pl.ds(sid * DPT, DPT)])
    plsc.subcore_barrier()
    pltpu.sync_copy(dst_ref.at[wid], idx)

    def body(j, carry):
        pltpu.sync_copy(ones, acc.at[idx.at[j]], add=True)
        return carry

    lax.fori_loop(0, NWIN, body, 0)
    plsc.subcore_barrier()
    pltpu.sync_copy(acc.at[pl.ds(sid * DPT, DPT)],
                    out_ref.at[cid, pl.ds(sid * DPT, DPT)])


_deg_kernel = pl.kernel(
    _deg_body,
    out_type=jax.ShapeDtypeStruct((NCORES, NPAD), jnp.float32),
    mesh=_mesh,
    compiler_params=pltpu.CompilerParams(use_tc_tiling_on_sc=False),
    scratch_types=[
        pltpu.VMEM((NWIN, W), jnp.int32),
        pltpu.VMEM((W,), jnp.float32),
        pltpu.VMEM((DPT,), jnp.float32),
        pltpu.VMEM_SHARED((NPAD,), jnp.float32),
    ],
)


def _agg_body(tab_ref, edge_ref, out_ref, ebuf, tab, acc, esem):
    cid = lax.axis_index("c")
    sid = lax.axis_index("s")
    pltpu.sync_copy(tab_ref.at[sid], tab)

    def zb(i, carry):
        acc[pl.ds(i * 16, 16)] = jnp.zeros((16,), jnp.float32)
        return carry

    lax.fori_loop(0, NPAD // 16, zb, 0)

    pltpu.async_copy(edge_ref.at[cid, 0], ebuf.at[0], esem.at[0])

    def chunk(k, carry):
        b = lax.rem(k, 2)
        nxt = 1 - b

        @pl.when(k + 1 < NCH)
        def _():
            pltpu.async_copy(edge_ref.at[cid, k + 1], ebuf.at[nxt],
                             esem.at[nxt])

        pltpu.make_async_copy(edge_ref.at[cid, k], ebuf.at[b],
                              esem.at[b]).wait()

        @plsc.parallel_loop(0, STEPS, 1, unroll=8)
        def step(i):
            e = ebuf[b, pl.ds(i * 16, 16)]
            s16 = jnp.bitwise_and(e, 0xFFFF)
            d16 = lax.shift_right_logical(e, 16)
            v = plsc.load_gather(tab, [s16])
            plsc.addupdate_scatter(acc, [d16], v)

        return carry

    lax.fori_loop(0, NCH, chunk, 0)
    pltpu.sync_copy(acc, out_ref.at[cid, sid])


_agg_kernel = pl.kernel(
    _agg_body,
    out_type=jax.ShapeDtypeStruct((NCORES, DH, NPAD), jnp.float32),
    mesh=_mesh,
    compiler_params=pltpu.CompilerParams(use_tc_tiling_on_sc=False,
                                         needs_layout_passes=False),
    scratch_types=[
        pltpu.VMEM((2, CH), jnp.int32),
        pltpu.VMEM((N,), jnp.float32),
        pltpu.VMEM((NPAD,), jnp.float32),
        pltpu.SemaphoreType.DMA((2,)),
    ],
)


def _tc1_body(x_ref, w1_ref, degp_ref, xwt_ref, z1t_ref, dinv_ref):
    deg = degp_ref[0:1, :N] + degp_ref[1:2, :N] + 1.0
    dinv = lax.rsqrt(deg)
    xwt = lax.dot_general(w1_ref[...], x_ref[...], (((0,), (1,)), ((), ())),
                          preferred_element_type=jnp.float32,
                          precision=lax.Precision.HIGHEST)
    xwt_ref[...] = xwt
    z1t_ref[...] = xwt * dinv
    dinv_ref[...] = dinv


def _tc1(x, w1, degp):
    return pl.pallas_call(
        _tc1_body,
        out_shape=[
            jax.ShapeDtypeStruct((DH, N), jnp.float32),
            jax.ShapeDtypeStruct((DH, N), jnp.float32),
            jax.ShapeDtypeStruct((1, N), jnp.float32),
        ],
    )(x, w1, degp)


def _tc2_body(agg_ref, xwt_ref, dinv_ref, w2_ref, b1_ref, hw2t_ref, z2t_ref):
    dinv = dinv_ref[...]
    agg = agg_ref[0, :, :N] + agg_ref[1, :, :N]
    pre = dinv * agg + dinv * dinv * xwt_ref[...] + b1_ref[...]
    h = jnp.maximum(pre, 0.0)
    hw2t = lax.dot_general(w2_ref[...], h, (((0,), (0,)), ((), ())),
                           preferred_element_type=jnp.float32,
                           precision=lax.Precision.HIGHEST)
    hw2t_ref[...] = hw2t
    z2t_ref[...] = hw2t * dinv


def _tc2(agg1, xwt, dinv, w2, b1):
    return pl.pallas_call(
        _tc2_body,
        out_shape=[
            jax.ShapeDtypeStruct((DH, N), jnp.float32),
            jax.ShapeDtypeStruct((DH, N), jnp.float32),
        ],
    )(agg1, xwt, dinv, w2, b1)


def _tc3_body(agg_ref, hw2t_ref, dinv_ref, b2_ref, eye_ref, out_ref):
    dinv = dinv_ref[...]
    lg = (dinv * (agg_ref[0, :, :N] + agg_ref[1, :, :N])
          + dinv * dinv * hw2t_ref[...] + b2_ref[...])
    m = jnp.max(lg, axis=0, keepdims=True)
    s = jnp.sum(jnp.exp(lg - m), axis=0, keepdims=True)
    lsm = lg - m - jnp.log(s)
    out_ref[...] = lax.dot_general(lsm, eye_ref[...], (((0,), (0,)), ((), ())),
                                   preferred_element_type=jnp.float32,
                                   precision=lax.Precision.HIGHEST)


def _tc3(agg2, hw2t, dinv, b2, eye):
    return pl.pallas_call(
        _tc3_body,
        out_shape=jax.ShapeDtypeStruct((N, DH), jnp.float32),
    )(agg2, hw2t, dinv, b2, eye)


def kernel(x, edge_index, W1, b1, W2, b2):
    src = edge_index[0].astype(jnp.int32)
    dst = edge_index[1].astype(jnp.int32)
    epack = jnp.bitwise_or(src, lax.shift_left(dst, 16))
    edges = epack.reshape(NCORES, NCH, CH)
    dstw = dst.reshape(NTILES, NWIN, W)
    degp = _deg_kernel(dstw)
    xwt, z1t, dinv = _tc1(x, W1, degp)
    agg1 = _agg_kernel(z1t, edges)
    hw2t, z2t = _tc2(agg1, xwt, dinv, W2, b1.reshape(DH, 1))
    agg2 = _agg_kernel(z2t, edges)
    return _tc3(agg2, hw2t, dinv, b2.reshape(DH, 1),
                jnp.eye(DH, dtype=jnp.float32))
